# baseline (speedup 1.0000x reference)
import functools

import jax
import jax.numpy as jnp
from jax import lax
from jax.experimental import pallas as pl
from jax.experimental.pallas import tpu as pltpu
from jax.experimental.pallas import tpu_sc as plsc

_NUM_CORES = 2
_NUM_SUBCORES = 16
_LANES = 16
_NW = _NUM_CORES * _NUM_SUBCORES


def _sample_body(pis_ref, g_ref, out_ref):
    z = pis_ref[...]
    g = g_ref[...]
    k = z.shape[1]
    m = jnp.max(z, axis=-1, keepdims=True)
    e = jnp.exp(z - m)
    p = e / jnp.sum(e, axis=-1, keepdims=True)
    v = g + jnp.log(p)
    vmax = jnp.max(v, axis=-1, keepdims=True)
    col = lax.broadcasted_iota(jnp.int32, z.shape, 1)
    idx = jnp.min(jnp.where(v == vmax, col, k), axis=-1, keepdims=True)
    row = lax.broadcasted_iota(jnp.int32, idx.shape, 0)
    out_ref[...] = row * k + idx


def _sc_body(means_hbm, lvs_hbm, ridx_hbm, eps_hbm, out_hbm,
             idx_v, mean_v, lv_v, eps_v, sem_m, sem_l, *, b_per_w, d):
    wid = lax.axis_index("s") * _NUM_CORES + lax.axis_index("c")
    base = wid * b_per_w
    pltpu.sync_copy(ridx_hbm.at[pl.ds(base, b_per_w)], idx_v)
    cm = pltpu.async_copy(means_hbm.at[idx_v], mean_v, sem_m)
    cl = pltpu.async_copy(lvs_hbm.at[idx_v], lv_v, sem_l)
    pltpu.sync_copy(eps_hbm.at[pl.ds(base, b_per_w)], eps_v)
    cm.wait()
    cl.wait()

    def row_body(r, carry):
        for j in range(d // _LANES):
            sl = pl.ds(j * _LANES, _LANES)
            mvec = mean_v[r, sl]
            lvec = lv_v[r, sl]
            evec = eps_v[r, sl]
            mean_v[r, sl] = mvec + jnp.exp(lvec * 0.5) * evec
        return carry

    lax.fori_loop(0, b_per_w, row_body, 0)
    pltpu.sync_copy(mean_v, out_hbm.at[pl.ds(base, b_per_w)])


def kernel(z_means, z_log_vars, z_pis):
    b, k, d = z_means.shape
    b_per_w = b // _NW

    g = jax.random.gumbel(jax.random.key(42), (b, k), jnp.float32)
    eps = jax.random.normal(jax.random.key(43), (b, d), jnp.float32)

    ridx = pl.pallas_call(
        _sample_body,
        out_shape=jax.ShapeDtypeStruct((b, 1), jnp.int32),
    )(z_pis, g).reshape(b)

    means2 = z_means.reshape(b * k, d)
    lvs2 = z_log_vars.reshape(b * k, d)

    sc_kernel = pl.kernel(
        functools.partial(_sc_body, b_per_w=b_per_w, d=d),
        out_type=jax.ShapeDtypeStruct((b, d), jnp.float32),
        mesh=plsc.VectorSubcoreMesh(core_axis_name="c", subcore_axis_name="s"),
        scratch_types=[
            pltpu.VMEM((b_per_w,), jnp.int32),
            pltpu.VMEM((b_per_w, d), jnp.float32),
            pltpu.VMEM((b_per_w, d), jnp.float32),
            pltpu.VMEM((b_per_w, d), jnp.float32),
            pltpu.SemaphoreType.DMA,
            pltpu.SemaphoreType.DMA,
        ],
    )
    return sc_kernel(means2, lvs2, ridx, eps)
